# baseline (speedup 1.0000x reference)
import functools

import jax
import jax.numpy as jnp
from jax import lax
from jax.experimental import pallas as pl
from jax.experimental.pallas import tpu as pltpu

_K = 16
_W = 1.0
_BIG = 3.0e38
_NEG_BIG = -3.0e38
_L = 128

_R1 = 512
_C1 = 2048
_R2 = 512

_HI = (((1,), (0,)), ((), ()))


def _extract(jm, cg3):
  cga, cgb, cgc = cg3
  r = jm.shape[0]
  g = cga.shape[0]
  hi = jm >> 7
  lo = jm & (_L - 1)
  oh_hi = (lax.broadcasted_iota(jnp.int32, (r, g), 1) == hi)
  ohb = oh_hi.astype(jnp.bfloat16)
  da = lax.dot_general(ohb, cga, _HI, preferred_element_type=jnp.float32)
  db = lax.dot_general(ohb, cgb, _HI, preferred_element_type=jnp.float32)
  dc = lax.dot_general(ohb, cgc, _HI, preferred_element_type=jnp.float32)
  grp = (da + db) + dc
  oh_lo = lax.broadcasted_iota(jnp.int32, (r, _L), 1) == lo
  outs = []
  for c in range(3):
    sl = grp[:, c * _L:(c + 1) * _L]
    outs.append(jnp.max(jnp.where(oh_lo, sl, _NEG_BIG), axis=1, keepdims=True))
  return jnp.concatenate(outs, axis=1)


def _chamfer_body(nc, p_ref, ct_ref, cga_ref, cgb_ref, cgc_ref, out_ref):
  p = p_ref[0]
  px, py, pz = p[:, 0:1], p[:, 1:2], p[:, 2:3]
  sq_p = px * px + py * py + pz * pz

  run_min = jnp.full((_R1, 1), _BIG, jnp.float32)
  run_jm = jnp.zeros((_R1, 1), jnp.int32)
  iota = lax.broadcasted_iota(jnp.int32, (_R1, _C1), 1)

  for c0 in range(0, nc, _C1):
    ct = ct_ref[0, :, pl.ds(c0, _C1)]
    cx, cy, cz = ct[0:1, :], ct[1:2, :], ct[2:3, :]
    sq_c = cx * cx + cy * cy + cz * cz
    cross = lax.dot_general(p, ct, _HI)
    d = sq_p + sq_c - 2.0 * cross
    m = jnp.min(d, axis=1, keepdims=True)
    jm = jnp.min(jnp.where(d == m, iota, _C1), axis=1, keepdims=True)
    upd = m < run_min
    run_min = jnp.where(upd, m, run_min)
    run_jm = jnp.where(upd, jm + c0, run_jm)

  out_ref[0] = _extract(run_jm, (cga_ref[0], cgb_ref[0], cgc_ref[0]))


def _knn_loss_body(n, p_ref, pt_ref, pga_ref, pgb_ref, pgc_ref,
                   m_ref, mt_ref, mga_ref, mgb_ref, mgc_ref, out_ref):
  b = pl.program_id(0)
  i = pl.program_id(1)

  negs, rows_c, cgs = [], [], []
  for full_ref, t_ref, ga_ref, gb_ref, gc_ref in (
      (p_ref, pt_ref, pga_ref, pgb_ref, pgc_ref),
      (m_ref, mt_ref, mga_ref, mgb_ref, mgc_ref)):
    t = t_ref[0]
    rows = full_ref[0, pl.ds(i * _R2, _R2), :]
    cx, cy, cz = t[0:1, :], t[1:2, :], t[2:3, :]
    rx, ry, rz = rows[:, 0:1], rows[:, 1:2], rows[:, 2:3]
    sq_r = rx * rx + ry * ry + rz * rz
    sq_c = cx * cx + cy * cy + cz * cz
    cross = lax.dot_general(rows, t, _HI)
    negs.append(2.0 * cross - sq_r - sq_c)
    rows_c.append(rows)
    cgs.append((ga_ref[0], gb_ref[0], gc_ref[0]))

  iota = lax.broadcasted_iota(jnp.int32, (_R2, n), 1)

  def step(neg, s):
    mx = jnp.max(neg, axis=1, keepdims=True)
    jm = jnp.min(jnp.where(neg == mx, iota, n), axis=1, keepdims=True)
    rel = _extract(jm, cgs[s]) - rows_c[s]
    return jnp.where(iota == jm, _NEG_BIG, neg), rel

  def body(_, carry):
    neg0, neg1, accv = carry
    neg0, rel0 = step(neg0, 0)
    neg1, rel1 = step(neg1, 1)
    dif = rel1 - rel0
    return neg0, neg1, accv + jnp.sum(dif * dif, axis=1, keepdims=True)

  accv0 = jnp.zeros((_R2, 1), jnp.float32)
  _, _, accv = lax.fori_loop(0, _K, body, (negs[0], negs[1], accv0))

  @pl.when(jnp.logical_and(b == 0, i == 0))
  def _():
    out_ref[0, 0] = jnp.float32(0.0)

  out_ref[0, 0] += jnp.sum(accv)


def _grouped(points):
  B, N, _ = points.shape
  g = N // _L
  cg = jnp.transpose(points.reshape(B, g, _L, 3), (0, 1, 3, 2)).reshape(
      B, g, 3 * _L)

  def trunc_bf16(x):
    t = lax.bitcast_convert_type(
        lax.bitcast_convert_type(x, jnp.uint32) & jnp.uint32(0xFFFF0000),
        jnp.float32)
    return t, t.astype(jnp.bfloat16)

  af, a = trunc_bf16(cg)
  r = cg - af
  bf, b = trunc_bf16(r)
  c = (r - bf).astype(jnp.bfloat16)
  return a, b, c


def kernel(completed, partial):
  B, Nc, _ = completed.shape
  _, Np, _ = partial.shape

  ct = jnp.swapaxes(completed, 1, 2)
  cga, cgb, cgc = _grouped(completed)

  matched = pl.pallas_call(
      functools.partial(_chamfer_body, Nc),
      grid=(B, Np // _R1),
      in_specs=[
          pl.BlockSpec((1, _R1, 3), lambda b, i: (b, i, 0)),
          pl.BlockSpec((1, 3, Nc), lambda b, i: (b, 0, 0)),
          pl.BlockSpec((1, Nc // _L, 3 * _L), lambda b, i: (b, 0, 0)),
          pl.BlockSpec((1, Nc // _L, 3 * _L), lambda b, i: (b, 0, 0)),
          pl.BlockSpec((1, Nc // _L, 3 * _L), lambda b, i: (b, 0, 0)),
      ],
      out_specs=pl.BlockSpec((1, _R1, 3), lambda b, i: (b, i, 0)),
      out_shape=jax.ShapeDtypeStruct((B, Np, 3), jnp.float32),
  )(partial, ct, cga, cgb, cgc)

  pt = jnp.swapaxes(partial, 1, 2)
  mt = jnp.swapaxes(matched, 1, 2)
  pga, pgb, pgc = _grouped(partial)
  mga, mgb, mgc = _grouped(matched)

  loss_sum = pl.pallas_call(
      functools.partial(_knn_loss_body, Np),
      grid=(B, Np // _R2),
      in_specs=[
          pl.BlockSpec((1, Np, 3), lambda b, i: (b, 0, 0)),
          pl.BlockSpec((1, 3, Np), lambda b, i: (b, 0, 0)),
          pl.BlockSpec((1, Np // _L, 3 * _L), lambda b, i: (b, 0, 0)),
          pl.BlockSpec((1, Np // _L, 3 * _L), lambda b, i: (b, 0, 0)),
          pl.BlockSpec((1, Np // _L, 3 * _L), lambda b, i: (b, 0, 0)),
          pl.BlockSpec((1, Np, 3), lambda b, i: (b, 0, 0)),
          pl.BlockSpec((1, 3, Np), lambda b, i: (b, 0, 0)),
          pl.BlockSpec((1, Np // _L, 3 * _L), lambda b, i: (b, 0, 0)),
          pl.BlockSpec((1, Np // _L, 3 * _L), lambda b, i: (b, 0, 0)),
          pl.BlockSpec((1, Np // _L, 3 * _L), lambda b, i: (b, 0, 0)),
      ],
      out_specs=pl.BlockSpec(memory_space=pltpu.SMEM),
      out_shape=jax.ShapeDtypeStruct((1, 1), jnp.float32),
  )(partial, pt, pga, pgb, pgc, matched, mt, mga, mgb, mgc)

  denom = B * Np * _K * 3
  return (_W / denom) * loss_sum[0, 0]


# speedup vs baseline: 1.0188x; 1.0188x over previous
import functools

import jax
import jax.numpy as jnp
from jax import lax
from jax.experimental import pallas as pl
from jax.experimental.pallas import tpu as pltpu

_K = 16
_W = 1.0
_BIG = 3.0e38
_NEG_BIG = -3.0e38
_L = 128

_R1 = 512
_C1 = 2048
_R2 = 1024

_HI = (((1,), (0,)), ((), ()))


def _extract(jm, cg3):
  cga, cgb, cgc = cg3
  r = jm.shape[0]
  g = cga.shape[0]
  hi = jm >> 7
  lo = jm & (_L - 1)
  oh_hi = (lax.broadcasted_iota(jnp.int32, (r, g), 1) == hi)
  ohb = oh_hi.astype(jnp.bfloat16)
  da = lax.dot_general(ohb, cga, _HI, preferred_element_type=jnp.float32)
  db = lax.dot_general(ohb, cgb, _HI, preferred_element_type=jnp.float32)
  dc = lax.dot_general(ohb, cgc, _HI, preferred_element_type=jnp.float32)
  grp = (da + db) + dc
  oh_lo = lax.broadcasted_iota(jnp.int32, (r, _L), 1) == lo
  outs = []
  for c in range(3):
    sl = grp[:, c * _L:(c + 1) * _L]
    outs.append(jnp.max(jnp.where(oh_lo, sl, _NEG_BIG), axis=1, keepdims=True))
  return jnp.concatenate(outs, axis=1)


def _chamfer_body(nc, p_ref, ct_ref, cga_ref, cgb_ref, cgc_ref, out_ref):
  p = p_ref[0]
  px, py, pz = p[:, 0:1], p[:, 1:2], p[:, 2:3]
  sq_p = px * px + py * py + pz * pz

  run_min = jnp.full((_R1, 1), _BIG, jnp.float32)
  run_jm = jnp.zeros((_R1, 1), jnp.int32)
  iota = lax.broadcasted_iota(jnp.int32, (_R1, _C1), 1)

  for c0 in range(0, nc, _C1):
    ct = ct_ref[0, :, pl.ds(c0, _C1)]
    cx, cy, cz = ct[0:1, :], ct[1:2, :], ct[2:3, :]
    sq_c = cx * cx + cy * cy + cz * cz
    cross = lax.dot_general(p, ct, _HI)
    d = sq_p + sq_c - 2.0 * cross
    m = jnp.min(d, axis=1, keepdims=True)
    jm = jnp.min(jnp.where(d == m, iota, _C1), axis=1, keepdims=True)
    upd = m < run_min
    run_min = jnp.where(upd, m, run_min)
    run_jm = jnp.where(upd, jm + c0, run_jm)

  out_ref[0] = _extract(run_jm, (cga_ref[0], cgb_ref[0], cgc_ref[0]))


def _knn_loss_body(n, p_ref, pt_ref, pga_ref, pgb_ref, pgc_ref,
                   m_ref, mt_ref, mga_ref, mgb_ref, mgc_ref, out_ref):
  b = pl.program_id(0)
  i = pl.program_id(1)

  negs, rows_c, cgs = [], [], []
  for full_ref, t_ref, ga_ref, gb_ref, gc_ref in (
      (p_ref, pt_ref, pga_ref, pgb_ref, pgc_ref),
      (m_ref, mt_ref, mga_ref, mgb_ref, mgc_ref)):
    t = t_ref[0]
    rows = full_ref[0, pl.ds(i * _R2, _R2), :]
    cx, cy, cz = t[0:1, :], t[1:2, :], t[2:3, :]
    rx, ry, rz = rows[:, 0:1], rows[:, 1:2], rows[:, 2:3]
    sq_r = rx * rx + ry * ry + rz * rz
    sq_c = cx * cx + cy * cy + cz * cz
    cross = lax.dot_general(rows, t, _HI)
    negs.append(2.0 * cross - sq_r - sq_c)
    rows_c.append(rows)
    cgs.append((ga_ref[0], gb_ref[0], gc_ref[0]))

  iota = lax.broadcasted_iota(jnp.int32, (_R2, n), 1)

  def step(neg, s):
    mx = jnp.max(neg, axis=1, keepdims=True)
    jm = jnp.min(jnp.where(neg == mx, iota, n), axis=1, keepdims=True)
    rel = _extract(jm, cgs[s]) - rows_c[s]
    return jnp.where(iota == jm, _NEG_BIG, neg), rel

  def body(_, carry):
    neg0, neg1, accv = carry
    neg0, rel0 = step(neg0, 0)
    neg1, rel1 = step(neg1, 1)
    dif = rel1 - rel0
    return neg0, neg1, accv + jnp.sum(dif * dif, axis=1, keepdims=True)

  accv0 = jnp.zeros((_R2, 1), jnp.float32)
  _, _, accv = lax.fori_loop(0, _K, body, (negs[0], negs[1], accv0))

  @pl.when(jnp.logical_and(b == 0, i == 0))
  def _():
    out_ref[0, 0] = jnp.float32(0.0)

  out_ref[0, 0] += jnp.sum(accv)


def _grouped(points):
  B, N, _ = points.shape
  g = N // _L
  cg = jnp.transpose(points.reshape(B, g, _L, 3), (0, 1, 3, 2)).reshape(
      B, g, 3 * _L)

  def trunc_bf16(x):
    t = lax.bitcast_convert_type(
        lax.bitcast_convert_type(x, jnp.uint32) & jnp.uint32(0xFFFF0000),
        jnp.float32)
    return t, t.astype(jnp.bfloat16)

  af, a = trunc_bf16(cg)
  r = cg - af
  bf, b = trunc_bf16(r)
  c = (r - bf).astype(jnp.bfloat16)
  return a, b, c


def kernel(completed, partial):
  B, Nc, _ = completed.shape
  _, Np, _ = partial.shape

  ct = jnp.swapaxes(completed, 1, 2)
  cga, cgb, cgc = _grouped(completed)

  matched = pl.pallas_call(
      functools.partial(_chamfer_body, Nc),
      grid=(B, Np // _R1),
      in_specs=[
          pl.BlockSpec((1, _R1, 3), lambda b, i: (b, i, 0)),
          pl.BlockSpec((1, 3, Nc), lambda b, i: (b, 0, 0)),
          pl.BlockSpec((1, Nc // _L, 3 * _L), lambda b, i: (b, 0, 0)),
          pl.BlockSpec((1, Nc // _L, 3 * _L), lambda b, i: (b, 0, 0)),
          pl.BlockSpec((1, Nc // _L, 3 * _L), lambda b, i: (b, 0, 0)),
      ],
      out_specs=pl.BlockSpec((1, _R1, 3), lambda b, i: (b, i, 0)),
      out_shape=jax.ShapeDtypeStruct((B, Np, 3), jnp.float32),
  )(partial, ct, cga, cgb, cgc)

  pt = jnp.swapaxes(partial, 1, 2)
  mt = jnp.swapaxes(matched, 1, 2)
  pga, pgb, pgc = _grouped(partial)
  mga, mgb, mgc = _grouped(matched)

  loss_sum = pl.pallas_call(
      functools.partial(_knn_loss_body, Np),
      grid=(B, Np // _R2),
      in_specs=[
          pl.BlockSpec((1, Np, 3), lambda b, i: (b, 0, 0)),
          pl.BlockSpec((1, 3, Np), lambda b, i: (b, 0, 0)),
          pl.BlockSpec((1, Np // _L, 3 * _L), lambda b, i: (b, 0, 0)),
          pl.BlockSpec((1, Np // _L, 3 * _L), lambda b, i: (b, 0, 0)),
          pl.BlockSpec((1, Np // _L, 3 * _L), lambda b, i: (b, 0, 0)),
          pl.BlockSpec((1, Np, 3), lambda b, i: (b, 0, 0)),
          pl.BlockSpec((1, 3, Np), lambda b, i: (b, 0, 0)),
          pl.BlockSpec((1, Np // _L, 3 * _L), lambda b, i: (b, 0, 0)),
          pl.BlockSpec((1, Np // _L, 3 * _L), lambda b, i: (b, 0, 0)),
          pl.BlockSpec((1, Np // _L, 3 * _L), lambda b, i: (b, 0, 0)),
      ],
      out_specs=pl.BlockSpec(memory_space=pltpu.SMEM),
      out_shape=jax.ShapeDtypeStruct((1, 1), jnp.float32),
  )(partial, pt, pga, pgb, pgc, matched, mt, mga, mgb, mgc)

  denom = B * Np * _K * 3
  return (_W / denom) * loss_sum[0, 0]


# speedup vs baseline: 1.0744x; 1.0545x over previous
import functools

import jax
import jax.numpy as jnp
from jax import lax
from jax.experimental import pallas as pl
from jax.experimental.pallas import tpu as pltpu

_K = 16
_W = 1.0
_BIG = 3.0e38
_NEG_BIG = -3.0e38
_L = 128

_R1 = 512
_C1 = 2048
_R2 = 1024

_HI = (((1,), (0,)), ((), ()))


def _extract(jm, cg3):
  cga, cgb, cgc = cg3
  r = jm.shape[0]
  g = cga.shape[0]
  hi = jm >> 7
  lo = jm & (_L - 1)
  oh_hi = (lax.broadcasted_iota(jnp.int32, (r, g), 1) == hi)
  ohb = oh_hi.astype(jnp.bfloat16)
  da = lax.dot_general(ohb, cga, _HI, preferred_element_type=jnp.float32)
  db = lax.dot_general(ohb, cgb, _HI, preferred_element_type=jnp.float32)
  dc = lax.dot_general(ohb, cgc, _HI, preferred_element_type=jnp.float32)
  grp = (da + db) + dc
  oh_lo = lax.broadcasted_iota(jnp.int32, (r, _L), 1) == lo
  outs = []
  for c in range(3):
    sl = grp[:, c * _L:(c + 1) * _L]
    outs.append(jnp.max(jnp.where(oh_lo, sl, _NEG_BIG), axis=1, keepdims=True))
  return jnp.concatenate(outs, axis=1)


def _chamfer_body(nc, p_ref, ct_ref, cga_ref, cgb_ref, cgc_ref, out_ref):
  p = p_ref[0]
  px, py, pz = p[:, 0:1], p[:, 1:2], p[:, 2:3]
  sq_p = px * px + py * py + pz * pz

  run_min = jnp.full((_R1, 1), _BIG, jnp.float32)
  run_jm = jnp.zeros((_R1, 1), jnp.int32)
  iota = lax.broadcasted_iota(jnp.int32, (_R1, _C1), 1)

  for c0 in range(0, nc, _C1):
    ct = ct_ref[0, :, pl.ds(c0, _C1)]
    cx, cy, cz = ct[0:1, :], ct[1:2, :], ct[2:3, :]
    sq_c = cx * cx + cy * cy + cz * cz
    cross = lax.dot_general(p, ct, _HI)
    d = sq_p + sq_c - 2.0 * cross
    m = jnp.min(d, axis=1, keepdims=True)
    jm = jnp.min(jnp.where(d == m, iota, _C1), axis=1, keepdims=True)
    upd = m < run_min
    run_min = jnp.where(upd, m, run_min)
    run_jm = jnp.where(upd, jm + c0, run_jm)

  out_ref[0] = _extract(run_jm, (cga_ref[0], cgb_ref[0], cgc_ref[0]))


def _knn_loss_body(n, p_ref, pt_ref, pga_ref, pgb_ref, pgc_ref,
                   m_ref, mt_ref, mga_ref, mgb_ref, mgc_ref, out_ref):
  b = pl.program_id(0)
  i = pl.program_id(1)

  negs, rows_c, cgs = [], [], []
  for full_ref, t_ref, ga_ref, gb_ref, gc_ref in (
      (p_ref, pt_ref, pga_ref, pgb_ref, pgc_ref),
      (m_ref, mt_ref, mga_ref, mgb_ref, mgc_ref)):
    t = t_ref[0]
    rows = full_ref[0, pl.ds(i * _R2, _R2), :]
    cx, cy, cz = t[0:1, :], t[1:2, :], t[2:3, :]
    rx, ry, rz = rows[:, 0:1], rows[:, 1:2], rows[:, 2:3]
    sq_r = rx * rx + ry * ry + rz * rz
    sq_c = cx * cx + cy * cy + cz * cz
    cross = lax.dot_general(rows, t, _HI)
    negs.append(2.0 * cross - sq_r - sq_c)
    rows_c.append(rows)
    cgs.append((ga_ref[0], gb_ref[0], gc_ref[0]))

  iota = lax.broadcasted_iota(jnp.int32, (_R2, n), 1)
  lane = lax.broadcasted_iota(jnp.int32, (_R2, _L), 1)

  def step(neg, s):
    m = jnp.full((_R2, _L), _NEG_BIG, jnp.float32)
    vidx = jnp.zeros((_R2, _L), jnp.int32)
    for v in range(n // _L):
      x_v = neg[:, v * _L:(v + 1) * _L]
      gt = x_v > m
      vidx = jnp.where(gt, v, vidx)
      m = jnp.maximum(x_v, m)
    mx = jnp.max(m, axis=1, keepdims=True)
    cand = vidx * _L + lane
    jm = jnp.min(jnp.where(m == mx, cand, n), axis=1, keepdims=True)
    rel = _extract(jm, cgs[s]) - rows_c[s]
    return jnp.where(iota == jm, _NEG_BIG, neg), rel

  def body(_, carry):
    neg0, neg1, accv = carry
    neg0, rel0 = step(neg0, 0)
    neg1, rel1 = step(neg1, 1)
    dif = rel1 - rel0
    return neg0, neg1, accv + jnp.sum(dif * dif, axis=1, keepdims=True)

  accv0 = jnp.zeros((_R2, 1), jnp.float32)
  _, _, accv = lax.fori_loop(0, _K, body, (negs[0], negs[1], accv0))

  @pl.when(jnp.logical_and(b == 0, i == 0))
  def _():
    out_ref[0, 0] = jnp.float32(0.0)

  out_ref[0, 0] += jnp.sum(accv)


def _grouped(points):
  B, N, _ = points.shape
  g = N // _L
  cg = jnp.transpose(points.reshape(B, g, _L, 3), (0, 1, 3, 2)).reshape(
      B, g, 3 * _L)

  def trunc_bf16(x):
    t = lax.bitcast_convert_type(
        lax.bitcast_convert_type(x, jnp.uint32) & jnp.uint32(0xFFFF0000),
        jnp.float32)
    return t, t.astype(jnp.bfloat16)

  af, a = trunc_bf16(cg)
  r = cg - af
  bf, b = trunc_bf16(r)
  c = (r - bf).astype(jnp.bfloat16)
  return a, b, c


def kernel(completed, partial):
  B, Nc, _ = completed.shape
  _, Np, _ = partial.shape

  ct = jnp.swapaxes(completed, 1, 2)
  cga, cgb, cgc = _grouped(completed)

  matched = pl.pallas_call(
      functools.partial(_chamfer_body, Nc),
      grid=(B, Np // _R1),
      in_specs=[
          pl.BlockSpec((1, _R1, 3), lambda b, i: (b, i, 0)),
          pl.BlockSpec((1, 3, Nc), lambda b, i: (b, 0, 0)),
          pl.BlockSpec((1, Nc // _L, 3 * _L), lambda b, i: (b, 0, 0)),
          pl.BlockSpec((1, Nc // _L, 3 * _L), lambda b, i: (b, 0, 0)),
          pl.BlockSpec((1, Nc // _L, 3 * _L), lambda b, i: (b, 0, 0)),
      ],
      out_specs=pl.BlockSpec((1, _R1, 3), lambda b, i: (b, i, 0)),
      out_shape=jax.ShapeDtypeStruct((B, Np, 3), jnp.float32),
  )(partial, ct, cga, cgb, cgc)

  pt = jnp.swapaxes(partial, 1, 2)
  mt = jnp.swapaxes(matched, 1, 2)
  pga, pgb, pgc = _grouped(partial)
  mga, mgb, mgc = _grouped(matched)

  loss_sum = pl.pallas_call(
      functools.partial(_knn_loss_body, Np),
      grid=(B, Np // _R2),
      in_specs=[
          pl.BlockSpec((1, Np, 3), lambda b, i: (b, 0, 0)),
          pl.BlockSpec((1, 3, Np), lambda b, i: (b, 0, 0)),
          pl.BlockSpec((1, Np // _L, 3 * _L), lambda b, i: (b, 0, 0)),
          pl.BlockSpec((1, Np // _L, 3 * _L), lambda b, i: (b, 0, 0)),
          pl.BlockSpec((1, Np // _L, 3 * _L), lambda b, i: (b, 0, 0)),
          pl.BlockSpec((1, Np, 3), lambda b, i: (b, 0, 0)),
          pl.BlockSpec((1, 3, Np), lambda b, i: (b, 0, 0)),
          pl.BlockSpec((1, Np // _L, 3 * _L), lambda b, i: (b, 0, 0)),
          pl.BlockSpec((1, Np // _L, 3 * _L), lambda b, i: (b, 0, 0)),
          pl.BlockSpec((1, Np // _L, 3 * _L), lambda b, i: (b, 0, 0)),
      ],
      out_specs=pl.BlockSpec(memory_space=pltpu.SMEM),
      out_shape=jax.ShapeDtypeStruct((1, 1), jnp.float32),
  )(partial, pt, pga, pgb, pgc, matched, mt, mga, mgb, mgc)

  denom = B * Np * _K * 3
  return (_W / denom) * loss_sum[0, 0]
